# Initial kernel scaffold; baseline (speedup 1.0000x reference)
import jax
import jax.numpy as jnp
from jax.experimental import pallas as pl


def kernel(h, u, v, etype, rel_weight):
    raise NotImplementedError("write your pallas kernel here")


# speedup vs baseline: 2.9139x; 2.9139x over previous
import functools

import jax
import jax.numpy as jnp
from jax import lax
from jax.experimental import pallas as pl
from jax.experimental.pallas import tpu as pltpu
from jax.experimental.pallas import tpu_sc as plsc

import numpy as np

N_NODES = 10000
N_EDGES = 320000
D = 128
N_ETYPES = 8

NC = 2
NS = 16
NW = NC * NS
E_PER_W = N_EDGES // NW
B = 80
NB = E_PER_W // B
LANES = 16
CHUNKS = D // LANES


def _rot(x, idx):
    return lax.gather(
        x, idx[:, None],
        dimension_numbers=lax.GatherDimensionNumbers(
            offset_dims=(), collapsed_slice_dims=(0,), start_index_map=(0,)),
        slice_sizes=(1,),
        mode=lax.GatherScatterMode.PROMISE_IN_BOUNDS)


def _distmul_body(h_hbm, u_hbm, v_hbm, t_hbm, w_hbm, out_hbm,
                  u_v, v_v, t_v, w_v, ru_v, rv_v, sc_v,
                  sem_u, sem_v):
    wid = lax.axis_index("s") * NC + lax.axis_index("c")
    base = wid * E_PER_W

    pltpu.sync_copy(u_hbm.at[pl.ds(base, E_PER_W)], u_v)
    pltpu.sync_copy(v_hbm.at[pl.ds(base, E_PER_W)], v_v)
    pltpu.sync_copy(t_hbm.at[pl.ds(base, E_PER_W)], t_v)
    pltpu.sync_copy(w_hbm, w_v)

    iota = lax.iota(jnp.int32, LANES)

    def block_body(g, carry):
        e0 = g * B
        cu = pltpu.async_copy(h_hbm.at[u_v.at[pl.ds(e0, B)]], ru_v, sem_u)
        cv = pltpu.async_copy(h_hbm.at[v_v.at[pl.ds(e0, B)]], rv_v, sem_v)
        cu.wait()
        cv.wait()

        rots = [jnp.bitwise_and(iota + k, LANES - 1) for k in (8, 4, 2, 1)]

        def group_body(gg, gcarry):
            eg = gg * LANES
            tvec = t_v[pl.ds(e0 + eg, LANES)]
            s = jnp.zeros((LANES,), jnp.float32)
            for j in range(LANES):
                t = tvec[j]
                e = eg + j
                acc = (ru_v[e, pl.ds(0, LANES)]
                       * rv_v[e, pl.ds(0, LANES)]
                       * w_v[t, pl.ds(0, LANES)])
                for k in range(1, CHUNKS):
                    acc = acc + (ru_v[e, pl.ds(k * LANES, LANES)]
                                 * rv_v[e, pl.ds(k * LANES, LANES)]
                                 * w_v[t, pl.ds(k * LANES, LANES)])
                r = acc
                for rv in rots:
                    r = r + _rot(r, rv)
                s = jnp.where(iota == j, r, s)
            sig = 1.0 / (1.0 + jnp.exp(-s))
            sc_v[pl.ds(e0 + eg, LANES)] = sig
            return gcarry

        lax.fori_loop(0, B // LANES, group_body, 0)
        return carry

    lax.fori_loop(0, NB, block_body, 0)
    pltpu.sync_copy(sc_v, out_hbm.at[pl.ds(base, E_PER_W)])


_distmul = functools.partial(
    pl.kernel,
    mesh=plsc.VectorSubcoreMesh(core_axis_name="c", subcore_axis_name="s"),
    out_type=jax.ShapeDtypeStruct((N_EDGES,), jnp.float32),
    scratch_types=[
        pltpu.VMEM((E_PER_W,), jnp.int32),
        pltpu.VMEM((E_PER_W,), jnp.int32),
        pltpu.VMEM((E_PER_W,), jnp.int32),
        pltpu.VMEM((N_ETYPES, D), jnp.float32),
        pltpu.VMEM((B, D), jnp.float32),
        pltpu.VMEM((B, D), jnp.float32),
        pltpu.VMEM((E_PER_W,), jnp.float32),
        pltpu.SemaphoreType.DMA,
        pltpu.SemaphoreType.DMA,
    ],
)(_distmul_body)


def kernel(h, u, v, etype, rel_weight):
    return _distmul(h, u.astype(jnp.int32), v.astype(jnp.int32),
                    etype.astype(jnp.int32), rel_weight)


# speedup vs baseline: 4.9927x; 1.7134x over previous
import functools

import jax
import jax.numpy as jnp
from jax import lax
from jax.experimental import pallas as pl
from jax.experimental.pallas import tpu as pltpu
from jax.experimental.pallas import tpu_sc as plsc

import numpy as np

N_NODES = 10000
N_EDGES = 320000
D = 128
N_ETYPES = 8

NC = 2
NS = 16
NW = NC * NS
E_PER_W = N_EDGES // NW
B = 80
NB = E_PER_W // B
LANES = 16
CHUNKS = D // LANES


def _rot(x, idx):
    return lax.gather(
        x, idx[:, None],
        dimension_numbers=lax.GatherDimensionNumbers(
            offset_dims=(), collapsed_slice_dims=(0,), start_index_map=(0,)),
        slice_sizes=(1,),
        mode=lax.GatherScatterMode.PROMISE_IN_BOUNDS)


def _distmul_body(h_hbm, u_hbm, v_hbm, t_hbm, w_hbm, out_hbm,
                  u_v, v_v, t_v, w_v, ru0_v, rv0_v, ru1_v, rv1_v, sc_v,
                  su0, sv0, su1, sv1):
    wid = lax.axis_index("s") * NC + lax.axis_index("c")
    base = wid * E_PER_W

    pltpu.sync_copy(u_hbm.at[pl.ds(base, E_PER_W)], u_v)
    pltpu.sync_copy(v_hbm.at[pl.ds(base, E_PER_W)], v_v)
    pltpu.sync_copy(t_hbm.at[pl.ds(base, E_PER_W)], t_v)
    pltpu.sync_copy(w_hbm, w_v)

    iota = lax.iota(jnp.int32, LANES)
    rots = [jnp.bitwise_and(iota + k, LANES - 1) for k in (8, 4, 2, 1)]
    bufs = ((ru0_v, rv0_v, su0, sv0), (ru1_v, rv1_v, su1, sv1))

    def start_block(g, ru, rv, sem_u, sem_v):
        e0 = g * B
        pltpu.async_copy(h_hbm.at[u_v.at[pl.ds(e0, B)]], ru, sem_u)
        pltpu.async_copy(h_hbm.at[v_v.at[pl.ds(e0, B)]], rv, sem_v)

    def wait_block(g, ru, rv, sem_u, sem_v):
        e0 = g * B
        pltpu.make_async_copy(h_hbm.at[u_v.at[pl.ds(e0, B)]], ru, sem_u).wait()
        pltpu.make_async_copy(h_hbm.at[v_v.at[pl.ds(e0, B)]], rv, sem_v).wait()

    def compute_block(g, ru_v, rv_v):
        e0 = g * B

        def group_body(gg, gcarry):
            eg = gg * LANES
            tvec = t_v[pl.ds(e0 + eg, LANES)]
            s = jnp.zeros((LANES,), jnp.float32)
            for j in range(LANES):
                t = tvec[j]
                e = eg + j
                acc = (ru_v[e, pl.ds(0, LANES)]
                       * rv_v[e, pl.ds(0, LANES)]
                       * w_v[t, pl.ds(0, LANES)])
                for k in range(1, CHUNKS):
                    acc = acc + (ru_v[e, pl.ds(k * LANES, LANES)]
                                 * rv_v[e, pl.ds(k * LANES, LANES)]
                                 * w_v[t, pl.ds(k * LANES, LANES)])
                r = acc
                for rv in rots:
                    r = r + _rot(r, rv)
                s = jnp.where(iota == j, r, s)
            sig = 1.0 / (1.0 + jnp.exp(-s))
            sc_v[pl.ds(e0 + eg, LANES)] = sig
            return gcarry

        lax.fori_loop(0, B // LANES, group_body, 0)

    start_block(0, *bufs[0])
    start_block(1, *bufs[1])

    def pair_body(i, carry):
        g0 = 2 * i
        wait_block(g0, *bufs[0])
        compute_block(g0, bufs[0][0], bufs[0][1])
        start_block(g0 + 2, *bufs[0])
        g1 = g0 + 1
        wait_block(g1, *bufs[1])
        compute_block(g1, bufs[1][0], bufs[1][1])

        @pl.when(g1 + 2 < NB)
        def _():
            start_block(g1 + 2, *bufs[1])

        return carry

    lax.fori_loop(0, (NB - 1) // 2, pair_body, 0)
    wait_block(NB - 1, *bufs[0])
    compute_block(NB - 1, bufs[0][0], bufs[0][1])

    pltpu.sync_copy(sc_v, out_hbm.at[pl.ds(base, E_PER_W)])


_distmul = functools.partial(
    pl.kernel,
    mesh=plsc.VectorSubcoreMesh(core_axis_name="c", subcore_axis_name="s"),
    out_type=jax.ShapeDtypeStruct((N_EDGES,), jnp.float32),
    scratch_types=[
        pltpu.VMEM((E_PER_W,), jnp.int32),
        pltpu.VMEM((E_PER_W,), jnp.int32),
        pltpu.VMEM((E_PER_W,), jnp.int32),
        pltpu.VMEM((N_ETYPES, D), jnp.float32),
        pltpu.VMEM((B, D), jnp.float32),
        pltpu.VMEM((B, D), jnp.float32),
        pltpu.VMEM((B, D), jnp.float32),
        pltpu.VMEM((B, D), jnp.float32),
        pltpu.VMEM((E_PER_W,), jnp.float32),
        pltpu.SemaphoreType.DMA,
        pltpu.SemaphoreType.DMA,
        pltpu.SemaphoreType.DMA,
        pltpu.SemaphoreType.DMA,
    ],
)(_distmul_body)


def kernel(h, u, v, etype, rel_weight):
    return _distmul(h, u.astype(jnp.int32), v.astype(jnp.int32),
                    etype.astype(jnp.int32), rel_weight)
